# baseline (speedup 1.0000x reference)
import functools

import jax
import jax.numpy as jnp
from jax import lax
from jax.experimental import pallas as pl
from jax.experimental.pallas import tpu as pltpu
from jax.experimental.pallas import tpu_sc as plsc

NUM_CORES = 2
NUM_SUBCORES = 16
NUM_WORKERS = NUM_CORES * NUM_SUBCORES
EMBED = 64
LANES = 128
CHUNK = 256
NBUF = 3
GRP = 128


def _pad128(n):
    return ((n + LANES - 1) // LANES) * LANES


def _make_sc_gather(batch, n_users, n_movies):
    bw = batch // NUM_WORKERS
    ngrp = bw // GRP
    mesh = plsc.VectorSubcoreMesh(core_axis_name="c", subcore_axis_name="s")

    @functools.partial(
        pl.kernel,
        mesh=mesh,
        compiler_params=pltpu.CompilerParams(needs_layout_passes=False),
        out_type=(
            jax.ShapeDtypeStruct((batch, LANES), jnp.float32),
            jax.ShapeDtypeStruct((batch, LANES), jnp.float32),
        ),
        scratch_types=[
            pltpu.VMEM((bw + 16,), jnp.int32),
            pltpu.VMEM((bw + 16,), jnp.int32),
            pltpu.VMEM((1, bw + 16), jnp.int32),
            pltpu.VMEM((1, bw + 16), jnp.int32),
            pltpu.VMEM((2 * NUM_WORKERS + 16,), jnp.int32),
            pltpu.VMEM((ngrp, GRP), jnp.int32),
            pltpu.VMEM((ngrp, GRP), jnp.int32),
            pltpu.VMEM((NBUF, EMBED, CHUNK), jnp.float32),
            pltpu.VMEM((bw + 8, LANES), jnp.float32),
            pltpu.SemaphoreType.DMA,
            pltpu.SemaphoreType.DMA,
        ],
    )
    def gather(us_hbm, up_hbm, ms_hbm, mp_hbm, ul_hbm, ml_hbm, nch_hbm,
               utabT_hbm, mtabT_hbm,
               xu_out, xm_out, uid_v, mid_v, ulist_v, mlist_v, nch_v,
               upos_v, mpos_v, chunks_v, rows_v, csem, osem):
        wid = lax.axis_index("s") * NUM_CORES + lax.axis_index("c")
        i0 = wid * bw
        pltpu.sync_copy(us_hbm.at[pl.ds(i0, bw)], uid_v.at[pl.ds(0, bw)])
        pltpu.sync_copy(ms_hbm.at[pl.ds(i0, bw)], mid_v.at[pl.ds(0, bw)])
        pltpu.sync_copy(ul_hbm.at[wid], ulist_v.at[:, pl.ds(0, bw)])
        pltpu.sync_copy(ml_hbm.at[wid], mlist_v.at[:, pl.ds(0, bw)])
        pltpu.sync_copy(nch_hbm, nch_v.at[pl.ds(0, 2 * NUM_WORKERS)])
        pltpu.sync_copy(up_hbm.at[wid], upos_v)
        pltpu.sync_copy(mp_hbm.at[wid], mpos_v)

        fidx = [lax.iota(jnp.int32, 16) + 16 * k for k in range(EMBED // 16)]

        def run_table(tabT_hbm, padded_cols, ids_v, list_v, nch_off,
                      pos_v, x_out):
            def sread(p):
                return ids_v[pl.ds(p, 16)][0]

            nchunk = nch_v[pl.ds(nch_off + wid, 16)][0]
            max_start = padded_cols - CHUNK

            def chunk_start(c):
                blk = list_v[0, pl.ds(c, 16)][0]
                return pl.multiple_of(
                    jnp.minimum(blk * CHUNK, max_start), LANES)

            def issue(c):
                pltpu.async_copy(
                    tabT_hbm.at[:, pl.ds(chunk_start(c), CHUNK)],
                    chunks_v.at[lax.rem(c, NBUF)], csem)

            def wait_chunk():
                pltpu.make_async_copy(
                    tabT_hbm.at[:, pl.ds(0, CHUNK)],
                    chunks_v.at[0], csem).wait()

            issue(0)
            for j in range(1, NBUF):
                @pl.when(nchunk > j)
                def _(j=j):
                    issue(j)
            wait_chunk()

            @pl.loop(0, nchunk - 1 + bw, init_carry=(0, 0))
            def _(k, carry):
                p, c = carry
                sid = sread(jnp.minimum(p, bw - 1))
                cs = chunk_start(c)
                extracting = jnp.logical_and(p < bw, sid < cs + CHUNK)
                col = jnp.clip(sid - cs, 0, CHUNK - 1)
                prow = jnp.where(extracting, p, bw)
                cbuf = chunks_v.at[lax.rem(c, NBUF)]
                for k16 in fidx:
                    v = plsc.load_gather(
                        cbuf, [k16, jnp.full((16,), col, jnp.int32)])
                    plsc.store_scatter(
                        rows_v, [jnp.full((16,), prow, jnp.int32), k16], v)
                adv = jnp.logical_and(jnp.logical_not(extracting),
                                      c + 1 < nchunk)

                @pl.when(adv)
                def _():
                    wait_chunk()

                @pl.when(jnp.logical_and(adv, c + NBUF < nchunk))
                def _():
                    issue(c + NBUF)

                return (p + extracting.astype(jnp.int32),
                        c + adv.astype(jnp.int32))
            for g in range(ngrp):
                pltpu.async_copy(
                    rows_v.at[pl.ds(g * GRP, GRP)],
                    x_out.at[pos_v.at[g]], osem)
            for _ in range(ngrp):
                pltpu.make_async_copy(
                    rows_v.at[pl.ds(0, GRP)],
                    x_out.at[pos_v.at[0]], osem).wait()

        run_table(utabT_hbm, _pad128(n_users), uid_v, ulist_v, 0,
                  upos_v, xu_out)
        run_table(mtabT_hbm, _pad128(n_movies), mid_v, mlist_v,
                  NUM_WORKERS, mpos_v, xm_out)

    return gather


def _mlp_body(u_ref, m_ref, w1u_ref, w1m_ref, b1_ref, w2_ref, b2_ref, o_ref):
    u = u_ref[...][:, :EMBED]
    m = m_ref[...][:, :EMBED]
    h = (jnp.dot(u, w1u_ref[...], preferred_element_type=jnp.float32)
         + jnp.dot(m, w1m_ref[...], preferred_element_type=jnp.float32)
         + b1_ref[...])
    h = jnp.maximum(h, 0.0)
    o_ref[...] = jnp.sum(h * w2_ref[...], axis=1) + b2_ref[0]


def _mlp(xu, xm, w1u, w1m, b1, w2, b2, block_rows=2048):
    batch = xu.shape[0]
    return pl.pallas_call(
        _mlp_body,
        grid=(batch // block_rows,),
        in_specs=[
            pl.BlockSpec((block_rows, LANES), lambda i: (i, 0)),
            pl.BlockSpec((block_rows, LANES), lambda i: (i, 0)),
            pl.BlockSpec((EMBED, EMBED), lambda i: (0, 0)),
            pl.BlockSpec((EMBED, EMBED), lambda i: (0, 0)),
            pl.BlockSpec((1, EMBED), lambda i: (0, 0)),
            pl.BlockSpec((1, EMBED), lambda i: (0, 0)),
            pl.BlockSpec(memory_space=pltpu.SMEM),
        ],
        out_specs=pl.BlockSpec((block_rows,), lambda i: (i,)),
        out_shape=jax.ShapeDtypeStruct((batch,), jnp.float32),
    )(xu, xm, w1u, w1m, b1, w2, b2)


def _chunk_list(sorted_ids, batch):
    bw = batch // NUM_WORKERS
    b2 = (sorted_ids // CHUNK).reshape(NUM_WORKERS, bw)
    flag = jnp.concatenate(
        [jnp.ones((NUM_WORKERS, 1), jnp.bool_), b2[:, 1:] != b2[:, :-1]],
        axis=1)
    pos = jnp.cumsum(flag.astype(jnp.int32), axis=1) - 1
    lst = jnp.zeros((NUM_WORKERS, bw), jnp.int32)
    lst = lst.at[jnp.arange(NUM_WORKERS)[:, None], pos].set(b2)
    return lst.reshape(NUM_WORKERS, 1, bw), pos[:, -1] + 1


def kernel(user_ids, movie_ids, user_table, movie_table, W1, b1, W2, b2):
    batch = user_ids.shape[0]
    uids = user_ids.astype(jnp.int32)
    mids = movie_ids.astype(jnp.int32)
    iota = lax.iota(jnp.int32, batch)
    us, up = lax.sort_key_val(uids, iota)
    ms, mp = lax.sort_key_val(mids, iota)
    up3 = up.reshape(NUM_WORKERS, batch // (NUM_WORKERS * GRP), GRP)
    mp3 = mp.reshape(NUM_WORKERS, batch // (NUM_WORKERS * GRP), GRP)
    ul, unch = _chunk_list(us, batch)
    ml, mnch = _chunk_list(ms, batch)
    nch = jnp.concatenate([unch, mnch]).astype(jnp.int32)
    xu, xm = _make_sc_gather(batch, user_table.shape[0],
                             movie_table.shape[0])(
        us, up3, ms, mp3, ul, ml, nch, user_table.T, movie_table.T)
    out = _mlp(xu, xm, W1[:, :EMBED].T, W1[:, EMBED:].T,
               b1.reshape(1, EMBED), W2, b2)
    return out.reshape(batch, 1)


# speedup vs baseline: 1.7247x; 1.7247x over previous
import functools

import jax
import jax.numpy as jnp
from jax import lax
from jax.experimental import pallas as pl
from jax.experimental.pallas import tpu as pltpu
from jax.experimental.pallas import tpu_sc as plsc

NUM_CORES = 2
NUM_SUBCORES = 16
NUM_WORKERS = NUM_CORES * NUM_SUBCORES
EMBED = 64
LANES = 128
CHUNK = 256
NBUF = 3
GRP = 128


def _pad128(n):
    return ((n + LANES - 1) // LANES) * LANES


def _make_sc_gather(batch, n_users, n_movies):
    bw = batch // NUM_WORKERS
    ngrp = bw // GRP
    mesh = plsc.VectorSubcoreMesh(core_axis_name="c", subcore_axis_name="s")

    @functools.partial(
        pl.kernel,
        mesh=mesh,
        compiler_params=pltpu.CompilerParams(needs_layout_passes=False),
        out_type=(
            jax.ShapeDtypeStruct((batch, LANES), jnp.float32),
            jax.ShapeDtypeStruct((batch, LANES), jnp.float32),
        ),
        scratch_types=[
            pltpu.VMEM((bw + 16,), jnp.int32),
            pltpu.VMEM((bw + 16,), jnp.int32),
            pltpu.VMEM((ngrp, GRP), jnp.int32),
            pltpu.VMEM((ngrp, GRP), jnp.int32),
            pltpu.VMEM((NBUF, EMBED, CHUNK), jnp.float32),
            pltpu.VMEM((bw + 8, LANES), jnp.float32),
            pltpu.SemaphoreType.DMA,
            pltpu.SemaphoreType.DMA,
        ],
    )
    def gather(us_hbm, up_hbm, ms_hbm, mp_hbm, utabT_hbm, mtabT_hbm,
               xu_out, xm_out, uid_v, mid_v, upos_v, mpos_v,
               chunks_v, rows_v, csem, osem):
        wid = lax.axis_index("s") * NUM_CORES + lax.axis_index("c")
        i0 = wid * bw
        pltpu.sync_copy(us_hbm.at[pl.ds(i0, bw)], uid_v.at[pl.ds(0, bw)])
        pltpu.sync_copy(ms_hbm.at[pl.ds(i0, bw)], mid_v.at[pl.ds(0, bw)])
        pltpu.sync_copy(up_hbm.at[wid], upos_v)
        pltpu.sync_copy(mp_hbm.at[wid], mpos_v)

        fidx = [lax.iota(jnp.int32, 16) + 16 * k for k in range(EMBED // 16)]

        def run_table(tabT_hbm, padded_cols, ids_v, pos_v, x_out):
            def sread(p):
                return ids_v[pl.ds(p, 16)][0]

            first = sread(0)
            last = sread(bw - 1)
            start = pl.multiple_of((first // LANES) * LANES, LANES)
            nchunk = (last - start) // CHUNK + 1
            max_start = padded_cols - CHUNK

            def chunk_start(c):
                return pl.multiple_of(
                    jnp.minimum(start + c * CHUNK, max_start), LANES)

            def issue(c):
                pltpu.async_copy(
                    tabT_hbm.at[:, pl.ds(chunk_start(c), CHUNK)],
                    chunks_v.at[lax.rem(c, NBUF)], csem)

            def wait_chunk():
                pltpu.make_async_copy(
                    tabT_hbm.at[:, pl.ds(0, CHUNK)],
                    chunks_v.at[0], csem).wait()

            issue(0)
            for j in range(1, NBUF):
                @pl.when(nchunk > j)
                def _(j=j):
                    issue(j)
            wait_chunk()

            @pl.loop(0, nchunk - 1 + bw, init_carry=(0, 0))
            def _(k, carry):
                p, c = carry
                sid = sread(jnp.minimum(p, bw - 1))
                cs = chunk_start(c)
                extracting = jnp.logical_and(p < bw, sid < cs + CHUNK)
                col = jnp.clip(sid - cs, 0, CHUNK - 1)
                prow = jnp.where(extracting, p, bw)
                cbuf = chunks_v.at[lax.rem(c, NBUF)]
                for k16 in fidx:
                    v = plsc.load_gather(
                        cbuf, [k16, jnp.full((16,), col, jnp.int32)])
                    plsc.store_scatter(
                        rows_v, [jnp.full((16,), prow, jnp.int32), k16], v)
                adv = jnp.logical_and(jnp.logical_not(extracting),
                                      c + 1 < nchunk)

                @pl.when(adv)
                def _():
                    wait_chunk()

                @pl.when(jnp.logical_and(adv, c + NBUF < nchunk))
                def _():
                    issue(c + NBUF)

                return (p + extracting.astype(jnp.int32),
                        c + adv.astype(jnp.int32))
            for g in range(ngrp):
                pltpu.async_copy(
                    rows_v.at[pl.ds(g * GRP, GRP)],
                    x_out.at[pos_v.at[g]], osem)
            for _ in range(ngrp):
                pltpu.make_async_copy(
                    rows_v.at[pl.ds(0, GRP)],
                    x_out.at[pos_v.at[0]], osem).wait()

        run_table(utabT_hbm, _pad128(n_users), uid_v, upos_v, xu_out)
        run_table(mtabT_hbm, _pad128(n_movies), mid_v, mpos_v, xm_out)

    return gather


def _mlp_body(u_ref, m_ref, w1u_ref, w1m_ref, b1_ref, w2_ref, b2_ref, o_ref):
    u = u_ref[...][:, :EMBED]
    m = m_ref[...][:, :EMBED]
    h = (jnp.dot(u, w1u_ref[...], preferred_element_type=jnp.float32)
         + jnp.dot(m, w1m_ref[...], preferred_element_type=jnp.float32)
         + b1_ref[...])
    h = jnp.maximum(h, 0.0)
    o_ref[...] = jnp.sum(h * w2_ref[...], axis=1) + b2_ref[0]


def _mlp(xu, xm, w1u, w1m, b1, w2, b2, block_rows=4096):
    batch = xu.shape[0]
    return pl.pallas_call(
        _mlp_body,
        grid=(batch // block_rows,),
        in_specs=[
            pl.BlockSpec((block_rows, LANES), lambda i: (i, 0)),
            pl.BlockSpec((block_rows, LANES), lambda i: (i, 0)),
            pl.BlockSpec((EMBED, EMBED), lambda i: (0, 0)),
            pl.BlockSpec((EMBED, EMBED), lambda i: (0, 0)),
            pl.BlockSpec((1, EMBED), lambda i: (0, 0)),
            pl.BlockSpec((1, EMBED), lambda i: (0, 0)),
            pl.BlockSpec(memory_space=pltpu.SMEM),
        ],
        out_specs=pl.BlockSpec((block_rows,), lambda i: (i,)),
        out_shape=jax.ShapeDtypeStruct((batch,), jnp.float32),
    )(xu, xm, w1u, w1m, b1, w2, b2)


def kernel(user_ids, movie_ids, user_table, movie_table, W1, b1, W2, b2):
    batch = user_ids.shape[0]
    uids = user_ids.astype(jnp.int32)
    mids = movie_ids.astype(jnp.int32)
    iota = lax.iota(jnp.int32, batch)
    us, up = lax.sort_key_val(uids, iota)
    ms, mp = lax.sort_key_val(mids, iota)
    up3 = up.reshape(NUM_WORKERS, batch // (NUM_WORKERS * GRP), GRP)
    mp3 = mp.reshape(NUM_WORKERS, batch // (NUM_WORKERS * GRP), GRP)
    xu, xm = _make_sc_gather(batch, user_table.shape[0],
                             movie_table.shape[0])(
        us, up3, ms, mp3, user_table.T, movie_table.T)
    out = _mlp(xu, xm, W1[:, :EMBED].T, W1[:, EMBED:].T,
               b1.reshape(1, EMBED), W2, b2)
    return out.reshape(batch, 1)


# speedup vs baseline: 1.8657x; 1.0818x over previous
import functools

import jax
import jax.numpy as jnp
from jax import lax
from jax.experimental import pallas as pl
from jax.experimental.pallas import tpu as pltpu
from jax.experimental.pallas import tpu_sc as plsc

NUM_CORES = 2
NUM_SUBCORES = 16
NUM_WORKERS = NUM_CORES * NUM_SUBCORES
EMBED = 64
LANES = 128
CHUNK = 128
NBUF = 6
GRP = 128


def _pad128(n):
    return ((n + LANES - 1) // LANES) * LANES


def _make_sc_gather(batch, n_users, n_movies):
    bw = batch // NUM_WORKERS
    ngrp = bw // GRP
    mesh = plsc.VectorSubcoreMesh(core_axis_name="c", subcore_axis_name="s")

    @functools.partial(
        pl.kernel,
        mesh=mesh,
        compiler_params=pltpu.CompilerParams(needs_layout_passes=False),
        out_type=(
            jax.ShapeDtypeStruct((batch, LANES), jnp.float32),
            jax.ShapeDtypeStruct((batch, LANES), jnp.float32),
        ),
        scratch_types=[
            pltpu.VMEM((bw + 16,), jnp.int32),
            pltpu.VMEM((bw + 16,), jnp.int32),
            pltpu.VMEM((ngrp, GRP), jnp.int32),
            pltpu.VMEM((ngrp, GRP), jnp.int32),
            pltpu.VMEM((NBUF, EMBED, CHUNK), jnp.float32),
            pltpu.VMEM((bw + 8, LANES), jnp.float32),
            pltpu.SemaphoreType.DMA,
            pltpu.SemaphoreType.DMA,
        ],
    )
    def gather(us_hbm, up_hbm, ms_hbm, mp_hbm, utabT_hbm, mtabT_hbm,
               xu_out, xm_out, uid_v, mid_v, upos_v, mpos_v,
               chunks_v, rows_v, csem, osem):
        wid = lax.axis_index("s") * NUM_CORES + lax.axis_index("c")
        i0 = wid * bw
        pltpu.sync_copy(us_hbm.at[pl.ds(i0, bw)], uid_v.at[pl.ds(0, bw)])
        pltpu.sync_copy(ms_hbm.at[pl.ds(i0, bw)], mid_v.at[pl.ds(0, bw)])
        pltpu.sync_copy(up_hbm.at[wid], upos_v)
        pltpu.sync_copy(mp_hbm.at[wid], mpos_v)

        fidx = [lax.iota(jnp.int32, 16) + 16 * k for k in range(EMBED // 16)]

        def run_table(tabT_hbm, padded_cols, ids_v, pos_v, x_out):
            def sread(p):
                return ids_v[pl.ds(p, 16)][0]

            first = sread(0)
            last = sread(bw - 1)
            start = pl.multiple_of((first // LANES) * LANES, LANES)
            nchunk = (last - start) // CHUNK + 1
            max_start = padded_cols - CHUNK

            def chunk_start(c):
                return pl.multiple_of(
                    jnp.minimum(start + c * CHUNK, max_start), LANES)

            def issue(c):
                pltpu.async_copy(
                    tabT_hbm.at[:, pl.ds(chunk_start(c), CHUNK)],
                    chunks_v.at[lax.rem(c, NBUF)], csem)

            def wait_chunk():
                pltpu.make_async_copy(
                    tabT_hbm.at[:, pl.ds(0, CHUNK)],
                    chunks_v.at[0], csem).wait()

            issue(0)
            for j in range(1, NBUF):
                @pl.when(nchunk > j)
                def _(j=j):
                    issue(j)
            wait_chunk()

            @pl.loop(0, nchunk - 1 + bw, init_carry=(0, 0))
            def _(k, carry):
                p, c = carry
                sid = sread(jnp.minimum(p, bw - 1))
                cs = chunk_start(c)
                extracting = jnp.logical_and(p < bw, sid < cs + CHUNK)
                col = jnp.clip(sid - cs, 0, CHUNK - 1)
                prow = jnp.where(extracting, p, bw)
                cbuf = chunks_v.at[lax.rem(c, NBUF)]
                for k16 in fidx:
                    v = plsc.load_gather(
                        cbuf, [k16, jnp.full((16,), col, jnp.int32)])
                    plsc.store_scatter(
                        rows_v, [jnp.full((16,), prow, jnp.int32), k16], v)
                adv = jnp.logical_and(jnp.logical_not(extracting),
                                      c + 1 < nchunk)

                @pl.when(adv)
                def _():
                    wait_chunk()

                @pl.when(jnp.logical_and(adv, c + NBUF < nchunk))
                def _():
                    issue(c + NBUF)

                return (p + extracting.astype(jnp.int32),
                        c + adv.astype(jnp.int32))
            for g in range(ngrp):
                pltpu.async_copy(
                    rows_v.at[pl.ds(g * GRP, GRP)],
                    x_out.at[pos_v.at[g]], osem)
            for _ in range(ngrp):
                pltpu.make_async_copy(
                    rows_v.at[pl.ds(0, GRP)],
                    x_out.at[pos_v.at[0]], osem).wait()

        run_table(utabT_hbm, _pad128(n_users), uid_v, upos_v, xu_out)
        run_table(mtabT_hbm, _pad128(n_movies), mid_v, mpos_v, xm_out)

    return gather


def _mlp_body(u_ref, m_ref, w1u_ref, w1m_ref, b1_ref, w2_ref, b2_ref, o_ref):
    u = u_ref[...][:, :EMBED]
    m = m_ref[...][:, :EMBED]
    h = (jnp.dot(u, w1u_ref[...], preferred_element_type=jnp.float32)
         + jnp.dot(m, w1m_ref[...], preferred_element_type=jnp.float32)
         + b1_ref[...])
    h = jnp.maximum(h, 0.0)
    o_ref[...] = jnp.sum(h * w2_ref[...], axis=1) + b2_ref[0]


def _mlp(xu, xm, w1u, w1m, b1, w2, b2, block_rows=4096):
    batch = xu.shape[0]
    return pl.pallas_call(
        _mlp_body,
        grid=(batch // block_rows,),
        in_specs=[
            pl.BlockSpec((block_rows, LANES), lambda i: (i, 0)),
            pl.BlockSpec((block_rows, LANES), lambda i: (i, 0)),
            pl.BlockSpec((EMBED, EMBED), lambda i: (0, 0)),
            pl.BlockSpec((EMBED, EMBED), lambda i: (0, 0)),
            pl.BlockSpec((1, EMBED), lambda i: (0, 0)),
            pl.BlockSpec((1, EMBED), lambda i: (0, 0)),
            pl.BlockSpec(memory_space=pltpu.SMEM),
        ],
        out_specs=pl.BlockSpec((block_rows,), lambda i: (i,)),
        out_shape=jax.ShapeDtypeStruct((batch,), jnp.float32),
    )(xu, xm, w1u, w1m, b1, w2, b2)


def kernel(user_ids, movie_ids, user_table, movie_table, W1, b1, W2, b2):
    batch = user_ids.shape[0]
    uids = user_ids.astype(jnp.int32)
    mids = movie_ids.astype(jnp.int32)
    iota = lax.iota(jnp.int32, batch)
    us, up = lax.sort_key_val(uids, iota)
    ms, mp = lax.sort_key_val(mids, iota)
    up3 = up.reshape(NUM_WORKERS, batch // (NUM_WORKERS * GRP), GRP)
    mp3 = mp.reshape(NUM_WORKERS, batch // (NUM_WORKERS * GRP), GRP)
    xu, xm = _make_sc_gather(batch, user_table.shape[0],
                             movie_table.shape[0])(
        us, up3, ms, mp3, user_table.T, movie_table.T)
    out = _mlp(xu, xm, W1[:, :EMBED].T, W1[:, EMBED:].T,
               b1.reshape(1, EMBED), W2, b2)
    return out.reshape(batch, 1)
